# baseline (speedup 1.0000x reference)
import functools
import jax
import jax.numpy as jnp
from jax import lax
from jax.experimental import pallas as pl
from jax.experimental.pallas import tpu as pltpu
from jax.experimental.pallas import tpu_sc as plsc

EPS = 1e-5
_NC, _NS, _L = 2, 16, 16
_NW = _NC * _NS


def _gather_body(ta, tb, src, dst, x_out, idxa_v, idxb_v, buf_v, sem, *, E):
    C = 2000
    per_w = E // _NW
    wid = lax.axis_index("s") * _NC + lax.axis_index("c")
    base_w = wid * per_w

    def step(i, _):
        base = base_w + i * C
        pltpu.sync_copy(src.at[pl.ds(base, C)], idxa_v)
        pltpu.sync_copy(dst.at[pl.ds(base, C)], idxb_v)
        pltpu.async_copy(ta.at[idxa_v], buf_v, sem).wait()
        pltpu.sync_copy(tb.at[idxb_v], buf_v, add=True)
        pltpu.sync_copy(buf_v, x_out.at[pl.ds(base, C)])
        return 0

    lax.fori_loop(0, per_w // C, step, 0)


def _stats1_body(x_ref, w1_ref, b1_ref, o_ref):
    x = x_ref[...].astype(jnp.bfloat16)
    w1 = w1_ref[...].astype(jnp.bfloat16)
    h1 = jnp.maximum(jnp.dot(x, w1, preferred_element_type=jnp.float32)
                     + b1_ref[...], 0.0)
    s = jnp.sum(h1, axis=0)
    q = jnp.sum(h1 * h1, axis=0)
    z = jnp.concatenate([s[None, :], q[None, :],
                         jnp.zeros((6, 32), jnp.float32)], axis=0)

    @pl.when(pl.program_id(0) == 0)
    def _():
        o_ref[...] = jnp.zeros_like(o_ref)

    o_ref[...] += z


def _h2_body(x_ref, w1_ref, b1_ref, m1_ref, s1_ref, g1_ref, t1_ref,
             w2_ref, b2_ref, h2t_ref, o_ref):
    x = x_ref[...].astype(jnp.bfloat16)
    w1 = w1_ref[...].astype(jnp.bfloat16)
    h1 = jnp.maximum(jnp.dot(x, w1, preferred_element_type=jnp.float32)
                     + b1_ref[...], 0.0)
    h1n = (h1 - m1_ref[...]) / s1_ref[...] * g1_ref[...] + t1_ref[...]
    w2 = w2_ref[...].astype(jnp.bfloat16)
    h2 = jnp.maximum(
        jnp.dot(h1n.astype(jnp.bfloat16), w2,
                preferred_element_type=jnp.float32) + b2_ref[...], 0.0)
    h2t_ref[...] = jnp.swapaxes(h2, 0, 1)
    s = jnp.sum(h2, axis=0)
    q = jnp.sum(h2 * h2, axis=0)
    z = jnp.concatenate([s[None, :], q[None, :],
                         jnp.zeros((6, 64), jnp.float32)], axis=0)

    @pl.when(pl.program_id(0) == 0)
    def _():
        o_ref[...] = jnp.zeros_like(o_ref)

    o_ref[...] += z


def _scatter_body(h2t, dst, p_out, acc_v, idx_v, val_v, sem0, sem1, *, E, M):
    C = 2000
    Eh = E // _NC
    nchunk = Eh // C
    c = lax.axis_index("c")
    s = lax.axis_index("s")
    sems = (sem0, sem1)

    def zstep(i, _):
        acc_v[pl.ds(i * _L, _L)] = jnp.zeros((_L,), jnp.float32)
        return 0
    lax.fori_loop(0, (M * 4) // _L, zstep, 0)

    def start_fetch(i, b):
        base = c * Eh + i * C
        pltpu.async_copy(dst.at[pl.ds(base, C)], idx_v.at[b], sems[b])
        for f in range(4):
            pltpu.async_copy(h2t.at[4 * s + f, pl.ds(base, C)],
                             val_v.at[b, f], sems[b])

    def wait_fetch(b):
        pltpu.make_async_copy(dst.at[pl.ds(0, C)], idx_v.at[b],
                              sems[b]).wait()
        for f in range(4):
            pltpu.make_async_copy(h2t.at[0, pl.ds(0, C)],
                                  val_v.at[b, f], sems[b]).wait()

    def process(b):
        def vstep(j, bad):
            idx4 = idx_v[b, pl.ds(j * _L, _L)] * 4
            for f in range(4):
                fidx = idx4 + f
                v = val_v[b, f, pl.ds(j * _L, _L)]
                g = plsc.load_gather(acc_v, [fidx])
                plsc.store_scatter(acc_v, [fidx], jnp.maximum(v, g))
                g2 = plsc.load_gather(acc_v, [fidx])
                bad = bad | (v > g2).astype(jnp.int32)
            return bad
        bad = lax.fori_loop(0, C // _L, vstep, jnp.zeros((_L,), jnp.int32))

        @pl.when(jnp.any(bad > 0))
        def _():
            def vfix(j, _):
                idx4 = idx_v[b, pl.ds(j * _L, _L)] * 4
                for f in range(4):
                    fidx = idx4 + f
                    v = val_v[b, f, pl.ds(j * _L, _L)]
                    g = plsc.load_gather(acc_v, [fidx])

                    def retry(nd):
                        plsc.store_scatter(acc_v, [fidx], v, mask=nd)
                        return v > plsc.load_gather(acc_v, [fidx])

                    lax.while_loop(lambda nd: jnp.any(nd), retry, v > g)
                return 0
            lax.fori_loop(0, C // _L, vfix, 0)

    start_fetch(0, 0)

    def step2(k, _):
        i = k * 2

        @pl.when(i + 1 < nchunk)
        def _():
            start_fetch(i + 1, 1)
        wait_fetch(0)
        process(0)

        @pl.when(i + 2 < nchunk)
        def _():
            start_fetch(i + 2, 0)

        @pl.when(i + 1 < nchunk)
        def _():
            wait_fetch(1)
            process(1)
        return 0

    lax.fori_loop(0, (nchunk + 1) // 2, step2, 0)
    pltpu.sync_copy(acc_v, p_out.at[c, s])


def _final_mm_body(pa_ref, pb_ref, m2_ref, s2_ref, g2_ref, t2_ref,
                   w3_ref, b3_ref, t_ref, o_ref):
    m = jnp.maximum(pa_ref[...], pb_ref[...])
    a = jnp.maximum((m - m2_ref[...]) / s2_ref[...] * g2_ref[...]
                    + t2_ref[...], 0.0)
    t = jnp.maximum(
        jnp.dot(a.astype(jnp.bfloat16), w3_ref[...].astype(jnp.bfloat16),
                preferred_element_type=jnp.float32) + b3_ref[...], 0.0)
    t_ref[...] = t
    s = jnp.sum(t, axis=0)
    q = jnp.sum(t * t, axis=0)
    z = jnp.concatenate([s[None, :], q[None, :],
                         jnp.zeros((6, 64), jnp.float32)], axis=0)

    @pl.when(pl.program_id(0) == 0)
    def _():
        o_ref[...] = jnp.zeros_like(o_ref)

    o_ref[...] += z


def _final_norm_body(t_ref, m3_ref, s3_ref, g3_ref, t3_ref, o_ref):
    o_ref[...] = ((t_ref[...] - m3_ref[...]) / s3_ref[...] * g3_ref[...]
                  + t3_ref[...])


def kernel(last_coors, last_features, current_coors, edge, W1, b1, g1, bt1,
           W2, b2, g2, bt2, W3, b3, g3, bt3):
    N = last_coors.shape[0]
    M = current_coors.shape[0]
    E = edge.shape[1]
    src = edge[1].astype(jnp.int32)
    dst = edge[0].astype(jnp.int32)

    tableA = jnp.concatenate(
        [last_features, last_coors, jnp.zeros((N, 9), jnp.float32)], axis=1)
    tableB = jnp.concatenate(
        [jnp.zeros((M, 4), jnp.float32), -current_coors,
         jnp.zeros((M, 9), jnp.float32)], axis=1)
    W1p = jnp.concatenate([W1, jnp.zeros((9, 32), jnp.float32)], axis=0)

    mesh = plsc.VectorSubcoreMesh(core_axis_name="c", subcore_axis_name="s")
    sc_params = pltpu.CompilerParams(use_tc_tiling_on_sc=False,
                                     needs_layout_passes=False)

    X = pl.kernel(
        functools.partial(_gather_body, E=E),
        out_type=jax.ShapeDtypeStruct((E, 16), jnp.float32),
        mesh=mesh,
        scratch_types=[
            pltpu.VMEM((2000,), jnp.int32),
            pltpu.VMEM((2000,), jnp.int32),
            pltpu.VMEM((2000, 16), jnp.float32),
            pltpu.SemaphoreType.DMA,
        ],
        compiler_params=sc_params,
    )(tableA, tableB, src, dst)

    Bc = 12800
    nblk = E // Bc
    stats1 = pl.pallas_call(
        _stats1_body,
        grid=(nblk,),
        in_specs=[pl.BlockSpec((Bc, 16), lambda i: (i, 0)),
                  pl.BlockSpec((16, 32), lambda i: (0, 0)),
                  pl.BlockSpec((1, 32), lambda i: (0, 0))],
        out_specs=pl.BlockSpec((8, 32), lambda i: (0, 0)),
        out_shape=jax.ShapeDtypeStruct((8, 32), jnp.float32),
    )(X, W1p, b1[None, :])

    mean1 = (stats1[0] / E)[None, :]
    var1 = stats1[1] / E - mean1[0] * mean1[0]
    sqrt1 = jnp.sqrt(var1 + EPS)[None, :]

    h2t, stats2 = pl.pallas_call(
        _h2_body,
        grid=(nblk,),
        in_specs=[pl.BlockSpec((Bc, 16), lambda i: (i, 0)),
                  pl.BlockSpec((16, 32), lambda i: (0, 0)),
                  pl.BlockSpec((1, 32), lambda i: (0, 0)),
                  pl.BlockSpec((1, 32), lambda i: (0, 0)),
                  pl.BlockSpec((1, 32), lambda i: (0, 0)),
                  pl.BlockSpec((1, 32), lambda i: (0, 0)),
                  pl.BlockSpec((1, 32), lambda i: (0, 0)),
                  pl.BlockSpec((32, 64), lambda i: (0, 0)),
                  pl.BlockSpec((1, 64), lambda i: (0, 0))],
        out_specs=[pl.BlockSpec((64, Bc), lambda i: (0, i)),
                   pl.BlockSpec((8, 64), lambda i: (0, 0))],
        out_shape=[jax.ShapeDtypeStruct((64, E), jnp.float32),
                   jax.ShapeDtypeStruct((8, 64), jnp.float32)],
    )(X, W1p, b1[None, :], mean1, sqrt1, g1[None, :], bt1[None, :],
      W2, b2[None, :])

    mean2 = (stats2[0] / E)[None, :]
    var2 = stats2[1] / E - mean2[0] * mean2[0]
    sqrt2 = jnp.sqrt(var2 + EPS)[None, :]

    P = pl.kernel(
        functools.partial(_scatter_body, E=E, M=M),
        out_type=jax.ShapeDtypeStruct((_NC, _NS, M * 4), jnp.float32),
        mesh=plsc.VectorSubcoreMesh(core_axis_name="c", subcore_axis_name="s"),
        scratch_types=[
            pltpu.VMEM((M * 4,), jnp.float32),
            pltpu.VMEM((2, 2000), jnp.int32),
            pltpu.VMEM((2, 4, 2000), jnp.float32),
            pltpu.SemaphoreType.DMA,
            pltpu.SemaphoreType.DMA,
        ],
        compiler_params=sc_params,
    )(h2t, dst)

    P4 = P.reshape(_NC, _NS, M, 4)
    Pn = jnp.concatenate([P4[:, si] for si in range(_NS)], axis=2)

    Br = 5000
    t, stats3 = pl.pallas_call(
        _final_mm_body,
        grid=(M // Br,),
        in_specs=[pl.BlockSpec((Br, 64), lambda i: (i, 0)),
                  pl.BlockSpec((Br, 64), lambda i: (i, 0)),
                  pl.BlockSpec((1, 64), lambda i: (0, 0)),
                  pl.BlockSpec((1, 64), lambda i: (0, 0)),
                  pl.BlockSpec((1, 64), lambda i: (0, 0)),
                  pl.BlockSpec((1, 64), lambda i: (0, 0)),
                  pl.BlockSpec((64, 64), lambda i: (0, 0)),
                  pl.BlockSpec((1, 64), lambda i: (0, 0))],
        out_specs=[pl.BlockSpec((Br, 64), lambda i: (i, 0)),
                   pl.BlockSpec((8, 64), lambda i: (0, 0))],
        out_shape=[jax.ShapeDtypeStruct((M, 64), jnp.float32),
                   jax.ShapeDtypeStruct((8, 64), jnp.float32)],
    )(Pn[0], Pn[1], mean2, sqrt2, g2[None, :], bt2[None, :], W3, b3[None, :])

    mean3 = (stats3[0] / M)[None, :]
    var3 = stats3[1] / M - mean3[0] * mean3[0]
    sqrt3 = jnp.sqrt(var3 + EPS)[None, :]

    out = pl.pallas_call(
        _final_norm_body,
        grid=(M // Br,),
        in_specs=[pl.BlockSpec((Br, 64), lambda i: (i, 0)),
                  pl.BlockSpec((1, 64), lambda i: (0, 0)),
                  pl.BlockSpec((1, 64), lambda i: (0, 0)),
                  pl.BlockSpec((1, 64), lambda i: (0, 0)),
                  pl.BlockSpec((1, 64), lambda i: (0, 0))],
        out_specs=pl.BlockSpec((Br, 64), lambda i: (i, 0)),
        out_shape=jax.ShapeDtypeStruct((M, 64), jnp.float32),
    )(t, mean3, sqrt3, g3[None, :], bt3[None, :])
    return out


# speedup vs baseline: 1.5297x; 1.5297x over previous
import functools
import jax
import jax.numpy as jnp
from jax import lax
from jax.experimental import pallas as pl
from jax.experimental.pallas import tpu as pltpu
from jax.experimental.pallas import tpu_sc as plsc

EPS = 1e-5
_NC, _NS, _L = 2, 16, 16
_NW = _NC * _NS


def _gather_body(ta, tb, src, dst, x_out, idxa_v, idxb_v, buf_v, sem, *, E):
    C = 2000
    per_w = E // _NW
    wid = lax.axis_index("s") * _NC + lax.axis_index("c")
    base_w = wid * per_w

    def step(i, _):
        base = base_w + i * C
        pltpu.sync_copy(src.at[pl.ds(base, C)], idxa_v)
        pltpu.sync_copy(dst.at[pl.ds(base, C)], idxb_v)
        pltpu.async_copy(ta.at[idxa_v], buf_v, sem).wait()
        pltpu.sync_copy(tb.at[idxb_v], buf_v, add=True)
        pltpu.sync_copy(buf_v, x_out.at[pl.ds(base, C)])
        return 0

    lax.fori_loop(0, per_w // C, step, 0)


def _stats1_body(x_ref, w1_ref, b1_ref, o_ref):
    x = x_ref[...].astype(jnp.bfloat16)
    w1 = w1_ref[...].astype(jnp.bfloat16)
    h1 = jnp.maximum(jnp.dot(x, w1, preferred_element_type=jnp.float32)
                     + b1_ref[...], 0.0)
    s = jnp.sum(h1, axis=0)
    q = jnp.sum(h1 * h1, axis=0)
    z = jnp.concatenate([s[None, :], q[None, :],
                         jnp.zeros((6, 32), jnp.float32)], axis=0)

    @pl.when(pl.program_id(0) == 0)
    def _():
        o_ref[...] = jnp.zeros_like(o_ref)

    o_ref[...] += z


def _h2_body(x_ref, w1_ref, b1_ref, m1_ref, s1_ref, g1_ref, t1_ref,
             w2_ref, b2_ref, h2t_ref, o_ref):
    x = x_ref[...].astype(jnp.bfloat16)
    w1 = w1_ref[...].astype(jnp.bfloat16)
    h1 = jnp.maximum(jnp.dot(x, w1, preferred_element_type=jnp.float32)
                     + b1_ref[...], 0.0)
    h1n = (h1 - m1_ref[...]) / s1_ref[...] * g1_ref[...] + t1_ref[...]
    w2 = w2_ref[...].astype(jnp.bfloat16)
    h2 = jnp.maximum(
        jnp.dot(h1n.astype(jnp.bfloat16), w2,
                preferred_element_type=jnp.float32) + b2_ref[...], 0.0)
    h2t_ref[...] = jnp.swapaxes(h2, 0, 1)
    s = jnp.sum(h2, axis=0)
    q = jnp.sum(h2 * h2, axis=0)
    z = jnp.concatenate([s[None, :], q[None, :],
                         jnp.zeros((6, 64), jnp.float32)], axis=0)

    @pl.when(pl.program_id(0) == 0)
    def _():
        o_ref[...] = jnp.zeros_like(o_ref)

    o_ref[...] += z


def _scatter_body(h2t, dst, p_out, acc_v, idx_v, val_v, sem0, sem1, *, E, M):
    C = 2000
    Eh = E // _NC
    nchunk = Eh // C
    c = lax.axis_index("c")
    s = lax.axis_index("s")
    sems = (sem0, sem1)

    def zstep(i, _):
        acc_v[pl.ds(i * _L, _L)] = jnp.zeros((_L,), jnp.float32)
        return 0
    lax.fori_loop(0, (M * 4) // _L, zstep, 0)

    def start_fetch(i, b):
        base = c * Eh + i * C
        pltpu.async_copy(dst.at[pl.ds(base, C)], idx_v.at[b], sems[b])
        for f in range(4):
            pltpu.async_copy(h2t.at[4 * s + f, pl.ds(base, C)],
                             val_v.at[b, f], sems[b])

    def wait_fetch(b):
        pltpu.make_async_copy(dst.at[pl.ds(0, C)], idx_v.at[b],
                              sems[b]).wait()
        for f in range(4):
            pltpu.make_async_copy(h2t.at[0, pl.ds(0, C)],
                                  val_v.at[b, f], sems[b]).wait()

    def process(b):
        def vstep(j, bad):
            idx4 = idx_v[b, pl.ds(j * _L, _L)] * 4
            for f in range(4):
                fidx = idx4 + f
                v = val_v[b, f, pl.ds(j * _L, _L)]
                g = plsc.load_gather(acc_v, [fidx])
                plsc.store_scatter(acc_v, [fidx], jnp.maximum(v, g))
                g2 = plsc.load_gather(acc_v, [fidx])
                bad = bad | (v > g2).astype(jnp.int32)
            return bad
        bad = lax.fori_loop(0, C // _L, vstep, jnp.zeros((_L,), jnp.int32))

        @pl.when(jnp.any(bad > 0))
        def _():
            def vfix(j, _):
                idx4 = idx_v[b, pl.ds(j * _L, _L)] * 4
                for f in range(4):
                    fidx = idx4 + f
                    v = val_v[b, f, pl.ds(j * _L, _L)]
                    g = plsc.load_gather(acc_v, [fidx])

                    def retry(nd):
                        plsc.store_scatter(acc_v, [fidx], v, mask=nd)
                        return v > plsc.load_gather(acc_v, [fidx])

                    lax.while_loop(lambda nd: jnp.any(nd), retry, v > g)
                return 0
            lax.fori_loop(0, C // _L, vfix, 0)

    start_fetch(0, 0)

    def step2(k, _):
        i = k * 2

        @pl.when(i + 1 < nchunk)
        def _():
            start_fetch(i + 1, 1)
        wait_fetch(0)
        process(0)

        @pl.when(i + 2 < nchunk)
        def _():
            start_fetch(i + 2, 0)

        @pl.when(i + 1 < nchunk)
        def _():
            wait_fetch(1)
            process(1)
        return 0

    lax.fori_loop(0, (nchunk + 1) // 2, step2, 0)
    pltpu.sync_copy(acc_v, p_out.at[c, s])


def _final_mm_body(pa_ref, pb_ref, m2_ref, s2_ref, g2_ref, t2_ref,
                   w3_ref, b3_ref, t_ref, o_ref):
    m = jnp.maximum(pa_ref[...], pb_ref[...])
    a = jnp.maximum((m - m2_ref[...]) / s2_ref[...] * g2_ref[...]
                    + t2_ref[...], 0.0)
    t = jnp.maximum(
        jnp.dot(a.astype(jnp.bfloat16), w3_ref[...].astype(jnp.bfloat16),
                preferred_element_type=jnp.float32) + b3_ref[...], 0.0)
    t_ref[...] = t
    s = jnp.sum(t, axis=0)
    q = jnp.sum(t * t, axis=0)
    z = jnp.concatenate([s[None, :], q[None, :],
                         jnp.zeros((6, 64), jnp.float32)], axis=0)

    @pl.when(pl.program_id(0) == 0)
    def _():
        o_ref[...] = jnp.zeros_like(o_ref)

    o_ref[...] += z


def _final_norm_body(t_ref, m3_ref, s3_ref, g3_ref, t3_ref, o_ref):
    o_ref[...] = ((t_ref[...] - m3_ref[...]) / s3_ref[...] * g3_ref[...]
                  + t3_ref[...])


def kernel(last_coors, last_features, current_coors, edge, W1, b1, g1, bt1,
           W2, b2, g2, bt2, W3, b3, g3, bt3):
    N = last_coors.shape[0]
    M = current_coors.shape[0]
    E = edge.shape[1]
    src = edge[1].astype(jnp.int32)
    dst = edge[0].astype(jnp.int32)

    tableA = jnp.concatenate(
        [last_features, last_coors, jnp.zeros((N, 9), jnp.float32)], axis=1)
    tableB = jnp.concatenate(
        [jnp.zeros((M, 4), jnp.float32), -current_coors,
         jnp.zeros((M, 9), jnp.float32)], axis=1)
    W1p = jnp.concatenate([W1, jnp.zeros((9, 32), jnp.float32)], axis=0)

    mesh = plsc.VectorSubcoreMesh(core_axis_name="c", subcore_axis_name="s")
    sc_params = pltpu.CompilerParams(use_tc_tiling_on_sc=False,
                                     needs_layout_passes=False)

    X = pl.kernel(
        functools.partial(_gather_body, E=E),
        out_type=jax.ShapeDtypeStruct((E, 16), jnp.float32),
        mesh=mesh,
        scratch_types=[
            pltpu.VMEM((2000,), jnp.int32),
            pltpu.VMEM((2000,), jnp.int32),
            pltpu.VMEM((2000, 16), jnp.float32),
            pltpu.SemaphoreType.DMA,
        ],
        compiler_params=sc_params,
    )(tableA, tableB, src, dst)

    Bc = 12800
    nblk = E // Bc
    stats1 = pl.pallas_call(
        _stats1_body,
        grid=(nblk,),
        in_specs=[pl.BlockSpec((Bc, 16), lambda i: (i, 0)),
                  pl.BlockSpec((16, 32), lambda i: (0, 0)),
                  pl.BlockSpec((1, 32), lambda i: (0, 0))],
        out_specs=pl.BlockSpec((8, 32), lambda i: (0, 0)),
        out_shape=jax.ShapeDtypeStruct((8, 32), jnp.float32),
    )(X, W1p, b1[None, :])

    mean1 = (stats1[0] / E)[None, :]
    var1 = stats1[1] / E - mean1[0] * mean1[0]
    sqrt1 = jnp.sqrt(var1 + EPS)[None, :]

    h2t, stats2 = pl.pallas_call(
        _h2_body,
        grid=(nblk,),
        in_specs=[pl.BlockSpec((Bc, 16), lambda i: (i, 0)),
                  pl.BlockSpec((16, 32), lambda i: (0, 0)),
                  pl.BlockSpec((1, 32), lambda i: (0, 0)),
                  pl.BlockSpec((1, 32), lambda i: (0, 0)),
                  pl.BlockSpec((1, 32), lambda i: (0, 0)),
                  pl.BlockSpec((1, 32), lambda i: (0, 0)),
                  pl.BlockSpec((1, 32), lambda i: (0, 0)),
                  pl.BlockSpec((32, 64), lambda i: (0, 0)),
                  pl.BlockSpec((1, 64), lambda i: (0, 0))],
        out_specs=[pl.BlockSpec((64, Bc), lambda i: (0, i)),
                   pl.BlockSpec((8, 64), lambda i: (0, 0))],
        out_shape=[jax.ShapeDtypeStruct((64, E), jnp.float32),
                   jax.ShapeDtypeStruct((8, 64), jnp.float32)],
    )(X, W1p, b1[None, :], mean1, sqrt1, g1[None, :], bt1[None, :],
      W2, b2[None, :])

    mean2 = (stats2[0] / E)[None, :]
    var2 = stats2[1] / E - mean2[0] * mean2[0]
    sqrt2 = jnp.sqrt(var2 + EPS)[None, :]

    P = pl.kernel(
        functools.partial(_scatter_body, E=E, M=M),
        out_type=jax.ShapeDtypeStruct((_NC, _NS, M * 4), jnp.float32),
        mesh=plsc.VectorSubcoreMesh(core_axis_name="c", subcore_axis_name="s"),
        scratch_types=[
            pltpu.VMEM((M * 4,), jnp.float32),
            pltpu.VMEM((2, 2000), jnp.int32),
            pltpu.VMEM((2, 4, 2000), jnp.float32),
            pltpu.SemaphoreType.DMA,
            pltpu.SemaphoreType.DMA,
        ],
        compiler_params=sc_params,
    )(h2t, dst)

    Pn = P.reshape(_NC, _NS, M, 4).transpose(0, 2, 1, 3).reshape(_NC, M, 64)

    Br = 5000
    t, stats3 = pl.pallas_call(
        _final_mm_body,
        grid=(M // Br,),
        in_specs=[pl.BlockSpec((Br, 64), lambda i: (i, 0)),
                  pl.BlockSpec((Br, 64), lambda i: (i, 0)),
                  pl.BlockSpec((1, 64), lambda i: (0, 0)),
                  pl.BlockSpec((1, 64), lambda i: (0, 0)),
                  pl.BlockSpec((1, 64), lambda i: (0, 0)),
                  pl.BlockSpec((1, 64), lambda i: (0, 0)),
                  pl.BlockSpec((64, 64), lambda i: (0, 0)),
                  pl.BlockSpec((1, 64), lambda i: (0, 0))],
        out_specs=[pl.BlockSpec((Br, 64), lambda i: (i, 0)),
                   pl.BlockSpec((8, 64), lambda i: (0, 0))],
        out_shape=[jax.ShapeDtypeStruct((M, 64), jnp.float32),
                   jax.ShapeDtypeStruct((8, 64), jnp.float32)],
    )(Pn[0], Pn[1], mean2, sqrt2, g2[None, :], bt2[None, :], W3, b3[None, :])

    mean3 = (stats3[0] / M)[None, :]
    var3 = stats3[1] / M - mean3[0] * mean3[0]
    sqrt3 = jnp.sqrt(var3 + EPS)[None, :]

    out = pl.pallas_call(
        _final_norm_body,
        grid=(M // Br,),
        in_specs=[pl.BlockSpec((Br, 64), lambda i: (i, 0)),
                  pl.BlockSpec((1, 64), lambda i: (0, 0)),
                  pl.BlockSpec((1, 64), lambda i: (0, 0)),
                  pl.BlockSpec((1, 64), lambda i: (0, 0)),
                  pl.BlockSpec((1, 64), lambda i: (0, 0))],
        out_specs=pl.BlockSpec((Br, 64), lambda i: (i, 0)),
        out_shape=jax.ShapeDtypeStruct((M, 64), jnp.float32),
    )(t, mean3, sqrt3, g3[None, :], bt3[None, :])
    return out
